# Initial kernel scaffold; baseline (speedup 1.0000x reference)
import jax
import jax.numpy as jnp
from jax.experimental import pallas as pl


def kernel(fea, index, W, b):
    raise NotImplementedError("write your pallas kernel here")


# speedup vs baseline: 14.1253x; 14.1253x over previous
import functools

import jax
import jax.numpy as jnp
from jax import lax
from jax.experimental import pallas as pl
from jax.experimental.pallas import tpu as pltpu
from jax.experimental.pallas import tpu_sc as plsc

N = 320000
D = 128
S = 10000
L = 16
NW = 32
CHUNK = N // NW
BN = 4000

_mesh = plsc.VectorSubcoreMesh(core_axis_name="c", subcore_axis_name="s")
_sc_params = pltpu.CompilerParams(needs_layout_passes=False)


def _gate_body(fea_ref, w_ref, gate_ref, max_ref):
    g = jnp.dot(fea_ref[...], w_ref[...], preferred_element_type=jnp.float32)
    gate_ref[...] = g
    bm = jnp.max(g)

    @pl.when(pl.program_id(0) == 0)
    def _():
        max_ref[...] = jnp.full((8, 128), bm, jnp.float32)

    @pl.when(pl.program_id(0) > 0)
    def _():
        max_ref[...] = jnp.maximum(max_ref[...], bm)


def _tc_gate(fea, W):
    return pl.pallas_call(
        _gate_body,
        grid=(N // BN,),
        in_specs=[
            pl.BlockSpec((BN, D), lambda i: (i, 0)),
            pl.BlockSpec((D, 1), lambda i: (0, 0)),
        ],
        out_specs=[
            pl.BlockSpec((BN, 1), lambda i: (i, 0)),
            pl.BlockSpec((8, 128), lambda i: (0, 0)),
        ],
        out_shape=[
            jax.ShapeDtypeStruct((N, 1), jnp.float32),
            jax.ShapeDtypeStruct((8, 128), jnp.float32),
        ],
    )(fea, W)


@functools.partial(
    pl.kernel,
    mesh=_mesh,
    out_type=[
        jax.ShapeDtypeStruct((N,), jnp.float32),
        jax.ShapeDtypeStruct((NW, S), jnp.float32),
    ],
    scratch_types=[
        pltpu.VMEM((CHUNK,), jnp.float32),
        pltpu.VMEM((CHUNK,), jnp.int32),
        pltpu.VMEM((CHUNK,), jnp.float32),
        pltpu.VMEM((S,), jnp.float32),
        pltpu.VMEM((L,), jnp.float32),
    ],
    compiler_params=_sc_params,
)
def _sc_exp_scatter(gate_hbm, idx_hbm, gmax_hbm, e_hbm, tab_hbm,
                    g_v, i_v, e_v, t_v, m_v):
    wid = lax.axis_index("s") * 2 + lax.axis_index("c")
    base = wid * CHUNK
    pltpu.sync_copy(gate_hbm.at[pl.ds(base, CHUNK)], g_v)
    pltpu.sync_copy(idx_hbm.at[pl.ds(base, CHUNK)], i_v)
    pltpu.sync_copy(gmax_hbm, m_v)
    m = m_v[...]

    def zero_body(j, carry):
        t_v[pl.ds(j * L, L)] = jnp.zeros((L,), jnp.float32)
        return carry

    lax.fori_loop(0, S // L, zero_body, 0)

    def body(j, carry):
        g = g_v[pl.ds(j * L, L)]
        ix = i_v[pl.ds(j * L, L)]
        e = jnp.exp(g - m)
        e_v[pl.ds(j * L, L)] = e
        plsc.addupdate_scatter(t_v, [ix], e)
        return carry

    lax.fori_loop(0, CHUNK // L, body, 0)
    pltpu.sync_copy(e_v, e_hbm.at[pl.ds(base, CHUNK)])
    pltpu.sync_copy(t_v, tab_hbm.at[wid])


def _reduce_body(tab_ref, tot_ref):
    tot_ref[...] = jnp.sum(tab_ref[...], axis=0, keepdims=True)


def _tc_reduce(tabs):
    return pl.pallas_call(
        _reduce_body,
        out_shape=jax.ShapeDtypeStruct((1, S), jnp.float32),
    )(tabs)


@functools.partial(
    pl.kernel,
    mesh=_mesh,
    out_type=jax.ShapeDtypeStruct((N,), jnp.float32),
    scratch_types=[
        pltpu.VMEM((CHUNK,), jnp.float32),
        pltpu.VMEM((CHUNK,), jnp.int32),
        pltpu.VMEM((S,), jnp.float32),
        pltpu.VMEM((CHUNK,), jnp.float32),
    ],
    compiler_params=_sc_params,
)
def _sc_apply(e_hbm, idx_hbm, tot_hbm, out_hbm, e_v, i_v, t_v, o_v):
    wid = lax.axis_index("s") * 2 + lax.axis_index("c")
    base = wid * CHUNK
    pltpu.sync_copy(e_hbm.at[pl.ds(base, CHUNK)], e_v)
    pltpu.sync_copy(idx_hbm.at[pl.ds(base, CHUNK)], i_v)
    pltpu.sync_copy(tot_hbm, t_v)

    def body(j, carry):
        e = e_v[pl.ds(j * L, L)]
        ix = i_v[pl.ds(j * L, L)]
        ssum = plsc.load_gather(t_v, [ix])
        o_v[pl.ds(j * L, L)] = e / (ssum + 1e-13)
        return carry

    lax.fori_loop(0, CHUNK // L, body, 0)
    pltpu.sync_copy(o_v, out_hbm.at[pl.ds(base, CHUNK)])


def kernel(fea, index, W, b):
    del b
    gate, gmax = _tc_gate(fea, W)
    gate1 = gate.reshape(N)
    gmax16 = gmax[0, :L]
    e, tabs = _sc_exp_scatter(gate1, index, gmax16)
    tot = _tc_reduce(tabs).reshape(S)
    out = _sc_apply(e, index, tot)
    return out.reshape(N, 1)


# speedup vs baseline: 22.9268x; 1.6231x over previous
import functools

import jax
import jax.numpy as jnp
from jax import lax
from jax.experimental import pallas as pl
from jax.experimental.pallas import tpu as pltpu
from jax.experimental.pallas import tpu_sc as plsc

N = 320000
D = 128
S = 10000
L = 16
NW = 32
CHUNK = N // NW
BN = 6400

_mesh = plsc.VectorSubcoreMesh(core_axis_name="c", subcore_axis_name="s")
_sc_params = pltpu.CompilerParams(needs_layout_passes=False)


def _gate_body(fea_ref, w_ref, gate_ref, max_ref):
    g = jnp.dot(fea_ref[...], w_ref[...], preferred_element_type=jnp.float32)
    g128 = g.reshape(1, BN // 128, 128)
    gate_ref[...] = g128
    bm = jnp.max(g128)

    @pl.when(pl.program_id(0) == 0)
    def _():
        max_ref[...] = jnp.full((8, 128), bm, jnp.float32)

    @pl.when(pl.program_id(0) > 0)
    def _():
        max_ref[...] = jnp.maximum(max_ref[...], bm)


def _tc_gate(fea, W):
    return pl.pallas_call(
        _gate_body,
        grid=(N // BN,),
        in_specs=[
            pl.BlockSpec((BN, D), lambda i: (i, 0)),
            pl.BlockSpec((D, 1), lambda i: (0, 0)),
        ],
        out_specs=[
            pl.BlockSpec((1, BN // 128, 128), lambda i: (i, 0, 0)),
            pl.BlockSpec((8, 128), lambda i: (0, 0)),
        ],
        out_shape=[
            jax.ShapeDtypeStruct((N // BN, BN // 128, 128), jnp.float32),
            jax.ShapeDtypeStruct((8, 128), jnp.float32),
        ],
    )(fea, W)


@functools.partial(
    pl.kernel,
    mesh=_mesh,
    out_type=[
        jax.ShapeDtypeStruct((N,), jnp.float32),
        jax.ShapeDtypeStruct((NW, S), jnp.float32),
    ],
    scratch_types=[
        pltpu.VMEM((CHUNK,), jnp.float32),
        pltpu.VMEM((CHUNK,), jnp.int32),
        pltpu.VMEM((CHUNK,), jnp.float32),
        pltpu.VMEM((S,), jnp.float32),
        pltpu.VMEM((L,), jnp.float32),
    ],
    compiler_params=_sc_params,
)
def _sc_exp_scatter(gate_hbm, idx_hbm, gmax_hbm, e_hbm, tab_hbm,
                    g_v, i_v, e_v, t_v, m_v):
    wid = lax.axis_index("s") * 2 + lax.axis_index("c")
    base = wid * CHUNK
    pltpu.sync_copy(gate_hbm.at[pl.ds(base, CHUNK)], g_v)
    pltpu.sync_copy(idx_hbm.at[pl.ds(base, CHUNK)], i_v)
    pltpu.sync_copy(gmax_hbm, m_v)
    m = m_v[...]

    def zero_body(j, carry):
        t_v[pl.ds(j * L, L)] = jnp.zeros((L,), jnp.float32)
        return carry

    lax.fori_loop(0, S // L, zero_body, 0)

    def body(j, carry):
        g = g_v[pl.ds(j * L, L)]
        ix = i_v[pl.ds(j * L, L)]
        e = jnp.exp(g - m)
        e_v[pl.ds(j * L, L)] = e
        plsc.addupdate_scatter(t_v, [ix], e)
        return carry

    lax.fori_loop(0, CHUNK // L, body, 0)
    pltpu.sync_copy(e_v, e_hbm.at[pl.ds(base, CHUNK)])
    pltpu.sync_copy(t_v, tab_hbm.at[wid])


def _reduce_body(tab_ref, tot_ref):
    tot_ref[...] = jnp.sum(tab_ref[...], axis=0)


def _tc_reduce(tabs):
    return pl.pallas_call(
        _reduce_body,
        out_shape=jax.ShapeDtypeStruct((S,), jnp.float32),
    )(tabs)


@functools.partial(
    pl.kernel,
    mesh=_mesh,
    out_type=jax.ShapeDtypeStruct((N,), jnp.float32),
    scratch_types=[
        pltpu.VMEM((CHUNK,), jnp.float32),
        pltpu.VMEM((CHUNK,), jnp.int32),
        pltpu.VMEM((S,), jnp.float32),
        pltpu.VMEM((CHUNK,), jnp.float32),
    ],
    compiler_params=_sc_params,
)
def _sc_apply(e_hbm, idx_hbm, tot_hbm, out_hbm, e_v, i_v, t_v, o_v):
    wid = lax.axis_index("s") * 2 + lax.axis_index("c")
    base = wid * CHUNK
    pltpu.sync_copy(e_hbm.at[pl.ds(base, CHUNK)], e_v)
    pltpu.sync_copy(idx_hbm.at[pl.ds(base, CHUNK)], i_v)
    pltpu.sync_copy(tot_hbm, t_v)

    def body(j, carry):
        e = e_v[pl.ds(j * L, L)]
        ix = i_v[pl.ds(j * L, L)]
        ssum = plsc.load_gather(t_v, [ix])
        o_v[pl.ds(j * L, L)] = e / (ssum + 1e-13)
        return carry

    lax.fori_loop(0, CHUNK // L, body, 0)
    pltpu.sync_copy(o_v, out_hbm.at[pl.ds(base, CHUNK)])


def kernel(fea, index, W, b):
    del b
    gate, gmax = _tc_gate(fea, W)
    gate1 = gate.reshape(N)
    gmax16 = gmax[0, :L]
    e, tabs = _sc_exp_scatter(gate1, index, gmax16)
    tot = _tc_reduce(tabs)
    return _sc_apply(e, index, tot).reshape(N, 1)


# speedup vs baseline: 25.7788x; 1.1244x over previous
import functools

import jax
import jax.numpy as jnp
from jax import lax
from jax.experimental import pallas as pl
from jax.experimental.pallas import tpu as pltpu
from jax.experimental.pallas import tpu_sc as plsc

N = 320000
D = 128
S = 10000
L = 16
NW = 32
CHUNK = N // NW
BN = 12800

_mesh = plsc.VectorSubcoreMesh(core_axis_name="c", subcore_axis_name="s")
_sc_params = pltpu.CompilerParams(needs_layout_passes=False)


def _gate_body(fea_ref, w_ref, gate_ref, max_ref):
    g = jnp.dot(fea_ref[...], w_ref[...], preferred_element_type=jnp.float32)
    g128 = g.reshape(1, BN // 128, 128)
    gate_ref[...] = g128
    bm = jnp.max(g128)

    @pl.when(pl.program_id(0) == 0)
    def _():
        max_ref[...] = jnp.full((8, 128), bm, jnp.float32)

    @pl.when(pl.program_id(0) > 0)
    def _():
        max_ref[...] = jnp.maximum(max_ref[...], bm)


def _tc_gate(fea, W):
    return pl.pallas_call(
        _gate_body,
        grid=(N // BN,),
        in_specs=[
            pl.BlockSpec((BN, D), lambda i: (i, 0)),
            pl.BlockSpec((D, 1), lambda i: (0, 0)),
        ],
        out_specs=[
            pl.BlockSpec((1, BN // 128, 128), lambda i: (i, 0, 0)),
            pl.BlockSpec((8, 128), lambda i: (0, 0)),
        ],
        out_shape=[
            jax.ShapeDtypeStruct((N // BN, BN // 128, 128), jnp.float32),
            jax.ShapeDtypeStruct((8, 128), jnp.float32),
        ],
    )(fea, W)


@functools.partial(
    pl.kernel,
    mesh=_mesh,
    out_type=jax.ShapeDtypeStruct((NW, S), jnp.float32),
    scratch_types=[
        pltpu.VMEM((CHUNK,), jnp.float32),
        pltpu.VMEM((CHUNK,), jnp.int32),
        pltpu.VMEM((S,), jnp.float32),
        pltpu.VMEM((L,), jnp.float32),
    ],
    compiler_params=_sc_params,
)
def _sc_exp_scatter(gate_hbm, idx_hbm, gmax_hbm, tab_hbm, g_v, i_v, t_v, m_v):
    wid = lax.axis_index("s") * 2 + lax.axis_index("c")
    base = wid * CHUNK
    pltpu.sync_copy(gate_hbm.at[pl.ds(base, CHUNK)], g_v)
    pltpu.sync_copy(idx_hbm.at[pl.ds(base, CHUNK)], i_v)
    pltpu.sync_copy(gmax_hbm, m_v)
    m = m_v[...]

    def zero_body(j, carry):
        t_v[pl.ds(j * L, L)] = jnp.zeros((L,), jnp.float32)
        return carry

    lax.fori_loop(0, S // L, zero_body, 0, unroll=8)

    def body(j, carry):
        g = g_v[pl.ds(j * L, L)]
        ix = i_v[pl.ds(j * L, L)]
        e = jnp.exp(g - m)
        plsc.addupdate_scatter(t_v, [ix], e)
        return carry

    lax.fori_loop(0, CHUNK // L, body, 0, unroll=8)
    pltpu.sync_copy(t_v, tab_hbm.at[wid])


def _reduce_body(tab_ref, tot_ref):
    tot_ref[...] = jnp.sum(tab_ref[...], axis=0)


def _tc_reduce(tabs):
    return pl.pallas_call(
        _reduce_body,
        out_shape=jax.ShapeDtypeStruct((S,), jnp.float32),
    )(tabs)


@functools.partial(
    pl.kernel,
    mesh=_mesh,
    out_type=jax.ShapeDtypeStruct((N,), jnp.float32),
    scratch_types=[
        pltpu.VMEM((CHUNK,), jnp.float32),
        pltpu.VMEM((CHUNK,), jnp.int32),
        pltpu.VMEM((S,), jnp.float32),
        pltpu.VMEM((CHUNK,), jnp.float32),
        pltpu.VMEM((L,), jnp.float32),
    ],
    compiler_params=_sc_params,
)
def _sc_apply(gate_hbm, idx_hbm, tot_hbm, gmax_hbm, out_hbm,
              g_v, i_v, t_v, o_v, m_v):
    wid = lax.axis_index("s") * 2 + lax.axis_index("c")
    base = wid * CHUNK
    pltpu.sync_copy(gate_hbm.at[pl.ds(base, CHUNK)], g_v)
    pltpu.sync_copy(idx_hbm.at[pl.ds(base, CHUNK)], i_v)
    pltpu.sync_copy(tot_hbm, t_v)
    pltpu.sync_copy(gmax_hbm, m_v)
    m = m_v[...]

    def body(j, carry):
        g = g_v[pl.ds(j * L, L)]
        ix = i_v[pl.ds(j * L, L)]
        e = jnp.exp(g - m)
        ssum = plsc.load_gather(t_v, [ix])
        o_v[pl.ds(j * L, L)] = e / (ssum + 1e-13)
        return carry

    lax.fori_loop(0, CHUNK // L, body, 0, unroll=8)
    pltpu.sync_copy(o_v, out_hbm.at[pl.ds(base, CHUNK)])


def kernel(fea, index, W, b):
    del b
    gate, gmax = _tc_gate(fea, W)
    gate1 = gate.reshape(N)
    gmax16 = gmax[0, :L]
    tabs = _sc_exp_scatter(gate1, index, gmax16)
    tot = _tc_reduce(tabs)
    return _sc_apply(gate1, index, tot, gmax16).reshape(N, 1)


# speedup vs baseline: 30.8393x; 1.1963x over previous
import functools

import jax
import jax.numpy as jnp
from jax import lax
from jax.experimental import pallas as pl
from jax.experimental.pallas import tpu as pltpu
from jax.experimental.pallas import tpu_sc as plsc

N = 320000
D = 128
S = 10000
L = 16
NW = 32
CHUNK = N // NW
BN = 32000

_mesh = plsc.VectorSubcoreMesh(core_axis_name="c", subcore_axis_name="s")
_sc_params = pltpu.CompilerParams(needs_layout_passes=False)


def _gate_body(fea_ref, w_ref, gate_ref, max_ref):
    g = jnp.dot(fea_ref[...], w_ref[...], preferred_element_type=jnp.float32)
    g128 = g.reshape(1, BN // 128, 128)
    gate_ref[...] = g128
    bm = jnp.max(g128)

    @pl.when(pl.program_id(0) == 0)
    def _():
        max_ref[...] = jnp.full((8, 128), bm, jnp.float32)

    @pl.when(pl.program_id(0) > 0)
    def _():
        max_ref[...] = jnp.maximum(max_ref[...], bm)


def _tc_gate(fea, W):
    return pl.pallas_call(
        _gate_body,
        grid=(N // BN,),
        in_specs=[
            pl.BlockSpec((BN, D), lambda i: (i, 0)),
            pl.BlockSpec((D, 1), lambda i: (0, 0)),
        ],
        out_specs=[
            pl.BlockSpec((1, BN // 128, 128), lambda i: (i, 0, 0)),
            pl.BlockSpec((8, 128), lambda i: (0, 0)),
        ],
        out_shape=[
            jax.ShapeDtypeStruct((N // BN, BN // 128, 128), jnp.float32),
            jax.ShapeDtypeStruct((8, 128), jnp.float32),
        ],
    )(fea, W)


@functools.partial(
    pl.kernel,
    mesh=_mesh,
    out_type=jax.ShapeDtypeStruct((NW, S), jnp.float32),
    scratch_types=[
        pltpu.VMEM((CHUNK,), jnp.float32),
        pltpu.VMEM((CHUNK,), jnp.int32),
        pltpu.VMEM((S,), jnp.float32),
        pltpu.VMEM((L,), jnp.float32),
    ],
    compiler_params=_sc_params,
)
def _sc_exp_scatter(gate_hbm, idx_hbm, gmax_hbm, tab_hbm, g_v, i_v, t_v, m_v):
    wid = lax.axis_index("s") * 2 + lax.axis_index("c")
    base = wid * CHUNK
    pltpu.sync_copy(gate_hbm.at[pl.ds(base, CHUNK)], g_v)
    pltpu.sync_copy(idx_hbm.at[pl.ds(base, CHUNK)], i_v)
    pltpu.sync_copy(gmax_hbm, m_v)
    m = m_v[...]

    @plsc.parallel_loop(0, S, step=L, unroll=8)
    def _zero(j):
        t_v[pl.ds(j, L)] = jnp.zeros((L,), jnp.float32)

    @plsc.parallel_loop(0, CHUNK, step=L, unroll=8)
    def _scatter(j):
        g = g_v[pl.ds(j, L)]
        ix = i_v[pl.ds(j, L)]
        e = jnp.exp(g - m)
        plsc.addupdate_scatter(t_v, [ix], e)

    pltpu.sync_copy(t_v, tab_hbm.at[wid])


def _reduce_body(tab_ref, tot_ref):
    tot_ref[...] = jnp.sum(tab_ref[...], axis=0)


def _tc_reduce(tabs):
    return pl.pallas_call(
        _reduce_body,
        out_shape=jax.ShapeDtypeStruct((S,), jnp.float32),
    )(tabs)


@functools.partial(
    pl.kernel,
    mesh=_mesh,
    out_type=jax.ShapeDtypeStruct((N,), jnp.float32),
    scratch_types=[
        pltpu.VMEM((CHUNK,), jnp.float32),
        pltpu.VMEM((CHUNK,), jnp.int32),
        pltpu.VMEM((S,), jnp.float32),
        pltpu.VMEM((CHUNK,), jnp.float32),
        pltpu.VMEM((L,), jnp.float32),
    ],
    compiler_params=_sc_params,
)
def _sc_apply(gate_hbm, idx_hbm, tot_hbm, gmax_hbm, out_hbm,
              g_v, i_v, t_v, o_v, m_v):
    wid = lax.axis_index("s") * 2 + lax.axis_index("c")
    base = wid * CHUNK
    pltpu.sync_copy(gate_hbm.at[pl.ds(base, CHUNK)], g_v)
    pltpu.sync_copy(idx_hbm.at[pl.ds(base, CHUNK)], i_v)
    pltpu.sync_copy(tot_hbm, t_v)
    pltpu.sync_copy(gmax_hbm, m_v)
    m = m_v[...]

    @plsc.parallel_loop(0, CHUNK, step=L, unroll=8)
    def _apply(j):
        g = g_v[pl.ds(j, L)]
        ix = i_v[pl.ds(j, L)]
        e = jnp.exp(g - m)
        ssum = plsc.load_gather(t_v, [ix])
        o_v[pl.ds(j, L)] = e / (ssum + 1e-13)

    pltpu.sync_copy(o_v, out_hbm.at[pl.ds(base, CHUNK)])


def kernel(fea, index, W, b):
    del b
    gate, gmax = _tc_gate(fea, W)
    gate1 = gate.reshape(N)
    gmax16 = gmax[0, :L]
    tabs = _sc_exp_scatter(gate1, index, gmax16)
    tot = _tc_reduce(tabs)
    return _sc_apply(gate1, index, tot, gmax16).reshape(N, 1)


# speedup vs baseline: 32.9601x; 1.0688x over previous
import functools

import jax
import jax.numpy as jnp
from jax import lax
from jax.experimental import pallas as pl
from jax.experimental.pallas import tpu as pltpu
from jax.experimental.pallas import tpu_sc as plsc

N = 320000
D = 128
S = 10000
L = 16
NW = 32
CHUNK = N // NW
BN = 32000

_mesh = plsc.VectorSubcoreMesh(core_axis_name="c", subcore_axis_name="s")
_sc_params = pltpu.CompilerParams(needs_layout_passes=False)


def _gate_body(fea_ref, w_ref, gate_ref, max_ref):
    g = jnp.dot(fea_ref[...], w_ref[...], preferred_element_type=jnp.float32)
    g128 = g.reshape(1, BN // 128, 128)
    gate_ref[...] = g128
    bm = jnp.max(g128)

    @pl.when(pl.program_id(0) == 0)
    def _():
        max_ref[...] = jnp.full((8, 128), bm, jnp.float32)

    @pl.when(pl.program_id(0) > 0)
    def _():
        max_ref[...] = jnp.maximum(max_ref[...], bm)


def _tc_gate(fea, W):
    return pl.pallas_call(
        _gate_body,
        grid=(N // BN,),
        in_specs=[
            pl.BlockSpec((BN, D), lambda i: (i, 0)),
            pl.BlockSpec((D, 1), lambda i: (0, 0)),
        ],
        out_specs=[
            pl.BlockSpec((1, BN // 128, 128), lambda i: (i, 0, 0)),
            pl.BlockSpec((8, 128), lambda i: (0, 0)),
        ],
        out_shape=[
            jax.ShapeDtypeStruct((N // BN, BN // 128, 128), jnp.float32),
            jax.ShapeDtypeStruct((8, 128), jnp.float32),
        ],
    )(fea, W)


@functools.partial(
    pl.kernel,
    mesh=_mesh,
    out_type=jax.ShapeDtypeStruct((NW, S), jnp.float32),
    scratch_types=[
        pltpu.VMEM((CHUNK,), jnp.float32),
        pltpu.VMEM((CHUNK,), jnp.int32),
        pltpu.VMEM((S,), jnp.float32),
        pltpu.VMEM((L,), jnp.float32),
    ],
    compiler_params=_sc_params,
)
def _sc_exp_scatter(gate_hbm, idx_hbm, gmax_hbm, tab_hbm, g_v, i_v, t_v, m_v):
    wid = lax.axis_index("s") * 2 + lax.axis_index("c")
    base = wid * CHUNK
    pltpu.sync_copy(gate_hbm.at[pl.ds(base, CHUNK)], g_v)
    pltpu.sync_copy(idx_hbm.at[pl.ds(base, CHUNK)], i_v)
    pltpu.sync_copy(gmax_hbm, m_v)
    m = m_v[...]

    @plsc.parallel_loop(0, S, step=L, unroll=8)
    def _zero(j):
        t_v[pl.ds(j, L)] = jnp.zeros((L,), jnp.float32)

    iota = lax.iota(jnp.int32, L)
    shift_r = jnp.maximum(iota - 1, 0)
    shift_l = jnp.minimum(iota + 1, L - 1)

    @plsc.parallel_loop(0, CHUNK, step=L, unroll=8)
    def _scatter(j):
        g = g_v[pl.ds(j, L)]
        ix = i_v[pl.ds(j, L)]
        e = jnp.exp(g - m)
        c = plsc.cumsum(e)
        cex = c - e
        start = (ix != ix[shift_r]) | (iota == 0)
        sp = plsc.cummax(jnp.where(start, iota, 0))
        run = c - cex[sp]
        end = (ix != ix[shift_l]) | (iota == L - 1)
        plsc.addupdate_scatter(t_v, [ix], run, mask=end)

    pltpu.sync_copy(t_v, tab_hbm.at[wid])


def _reduce_body(tab_ref, tot_ref):
    tot_ref[...] = jnp.sum(tab_ref[...], axis=0)


def _tc_reduce(tabs):
    return pl.pallas_call(
        _reduce_body,
        out_shape=jax.ShapeDtypeStruct((S,), jnp.float32),
    )(tabs)


@functools.partial(
    pl.kernel,
    mesh=_mesh,
    out_type=jax.ShapeDtypeStruct((N,), jnp.float32),
    scratch_types=[
        pltpu.VMEM((CHUNK,), jnp.float32),
        pltpu.VMEM((CHUNK,), jnp.int32),
        pltpu.VMEM((S,), jnp.float32),
        pltpu.VMEM((CHUNK,), jnp.float32),
        pltpu.VMEM((L,), jnp.float32),
    ],
    compiler_params=_sc_params,
)
def _sc_apply(gate_hbm, idx_hbm, tot_hbm, gmax_hbm, out_hbm,
              g_v, i_v, t_v, o_v, m_v):
    wid = lax.axis_index("s") * 2 + lax.axis_index("c")
    base = wid * CHUNK
    pltpu.sync_copy(gate_hbm.at[pl.ds(base, CHUNK)], g_v)
    pltpu.sync_copy(idx_hbm.at[pl.ds(base, CHUNK)], i_v)
    pltpu.sync_copy(tot_hbm, t_v)
    pltpu.sync_copy(gmax_hbm, m_v)
    m = m_v[...]

    @plsc.parallel_loop(0, CHUNK, step=L, unroll=8)
    def _apply(j):
        g = g_v[pl.ds(j, L)]
        ix = i_v[pl.ds(j, L)]
        e = jnp.exp(g - m)
        ssum = plsc.load_gather(t_v, [ix])
        o_v[pl.ds(j, L)] = e / (ssum + 1e-13)

    pltpu.sync_copy(o_v, out_hbm.at[pl.ds(base, CHUNK)])


def kernel(fea, index, W, b):
    del b
    gate, gmax = _tc_gate(fea, W)
    gate1 = gate.reshape(N)
    gmax16 = gmax[0, :L]
    tabs = _sc_exp_scatter(gate1, index, gmax16)
    tot = _tc_reduce(tabs)
    return _sc_apply(gate1, index, tot, gmax16).reshape(N, 1)
